# baseline (speedup 1.0000x reference)
import math

import jax
import jax.numpy as jnp
import numpy as np
from jax.experimental import pallas as pl

N = 10000
K = 16
E = N * K
CUTOFF = 3.0
RMIN = 3.5
NUM_RADIAL = 5
NUM_3BODY = 12
ETA = 4.0
N_DESC = NUM_RADIAL + NUM_3BODY
HIDDEN = 128

BN = 512
GRID = (N + BN - 1) // BN
NPAD = GRID * BN
FB = K * BN

_MU = np.linspace(-1.0, 1.0, NUM_3BODY)
_MU_SCALE = np.exp(-ETA * _MU * _MU)
_U_COEF = np.float32(4.0 * ETA / (NUM_3BODY - 1))
_RBF_PREF = np.float32(math.sqrt(2.0 / CUTOFF))

_HI = jax.lax.Precision.DEFAULT


def _pair_nn_block(r_ref, w1t_ref, w2t_ref, b2_ref, out_ref):
    a3 = r_ref[...].reshape(K, 3, BN)
    rx = a3[:, 0, :]
    ry = a3[:, 1, :]
    rz = a3[:, 2, :]

    r = jnp.sqrt(rx * rx + ry * ry + rz * rz)
    rs = jnp.maximum(r, 1e-12)
    inv = 1.0 / rs

    x = np.float32(np.pi / CUTOFF) * r
    s1 = jnp.sin(x)
    c1 = jnp.cos(x)

    c3 = (4.0 * c1 * c1 - 3.0) * c1
    fc = jnp.where(r > RMIN, 0.5 - 0.5 * (2.0 * c3 * c3 - 1.0), 1.0)

    pref = _RBF_PREF * inv * fc
    feats = [(pref * s1).reshape(1, FB)]
    s, c = s1, c1
    for _ in range(NUM_RADIAL - 1):
        s, c = s * c1 + c * s1, c * c1 - s * s1
        feats.append((pref * s).reshape(1, FB))

    fcrik = 0.5 + 0.5 * c1
    ux = (rx / rs).astype(jnp.bfloat16).astype(jnp.float32)
    uy = (ry / rs).astype(jnp.bfloat16).astype(jnp.float32)
    uz = (rz / rs).astype(jnp.bfloat16).astype(jnp.float32)
    cos3 = (
        ux[:, None, :] * ux[None, :, :]
        + uy[:, None, :] * uy[None, :, :]
        + uz[:, None, :] * uz[None, :, :]
    )
    il = jax.lax.broadcasted_iota(jnp.int32, (K, K, 1), 0)
    ik = jax.lax.broadcasted_iota(jnp.int32, (K, K, 1), 1)
    cos3 = jnp.where(il == ik, 0.0, cos3)

    p = jnp.exp(np.float32(-ETA) * cos3 * (cos3 + 2.0)) * fcrik[:, None, :]
    u = jnp.exp(_U_COEF * cos3)
    for m in range(NUM_3BODY):
        feats.append(
            (np.float32(_MU_SCALE[m]) * jnp.sum(p, axis=0)).reshape(1, FB))
        if m < NUM_3BODY - 1:
            p = p * u

    feats.append(jnp.ones((2, FB), dtype=jnp.float32))
    dmat = jnp.concatenate(feats, axis=0)
    pre = jnp.dot(w1t_ref[...], dmat.astype(jnp.bfloat16),
                  preferred_element_type=jnp.float32)
    h = jnp.tanh(pre)
    e = jnp.dot(w2t_ref[...], h.astype(jnp.bfloat16),
                preferred_element_type=jnp.float32) + b2_ref[0, 0]
    out_ref[...] = jnp.transpose(e.reshape(K, BN))


def kernel(elems, descriptors, beta, energy, rij, unique_i, unique_j,
           tag_i, tag_j, W1, b1, W2, b2):
    b1hi = b1.astype(jnp.bfloat16)
    b1lo = (b1 - b1hi.astype(jnp.float32)).astype(jnp.bfloat16)
    w1c = jnp.concatenate([W1.T.astype(jnp.bfloat16),
                           b1hi[:, None], b1lo[:, None]], axis=1)
    out = pl.pallas_call(
        _pair_nn_block,
        grid=(GRID,),
        in_specs=[
            pl.BlockSpec((K * 3, BN), lambda i: (0, i)),
            pl.BlockSpec((HIDDEN, N_DESC + 2), lambda i: (0, 0)),
            pl.BlockSpec((1, HIDDEN), lambda i: (0, 0)),
            pl.BlockSpec((1, 1), lambda i: (0, 0)),
        ],
        out_specs=pl.BlockSpec((BN, K), lambda i: (i, 0)),
        out_shape=jax.ShapeDtypeStruct((N, K), jnp.float32),
    )(rij.reshape(N, K * 3).T, w1c,
      W2.T.astype(jnp.bfloat16), b2.reshape(1, 1))
    return out.reshape(E, 1)


# speedup vs baseline: 1.9472x; 1.9472x over previous
import math

import jax
import jax.numpy as jnp
import numpy as np
from jax.experimental import pallas as pl

N = 10000
K = 16
E = N * K
CUTOFF = 3.0
RMIN = 3.5
NUM_RADIAL = 5
NUM_3BODY = 12
ETA = 4.0
N_DESC = NUM_RADIAL + NUM_3BODY
HIDDEN = 128

BN = 512
GRID = (N + BN - 1) // BN
NPAD = GRID * BN
FB = K * BN

_MU = np.linspace(-1.0, 1.0, NUM_3BODY)
_MU_SCALE = np.exp(-ETA * _MU * _MU)
_U_COEF = np.float32(4.0 * ETA / (NUM_3BODY - 1))
_RBF_PREF = np.float32(math.sqrt(2.0 / CUTOFF))

_HI = jax.lax.Precision.DEFAULT


def _pair_nn_block(r_ref, w1t_ref, w2t_ref, b2_ref, out_ref):
    rx = r_ref[0]
    ry = r_ref[1]
    rz = r_ref[2]

    r = jnp.sqrt(rx * rx + ry * ry + rz * rz)
    rs = jnp.maximum(r, 1e-12)
    inv = 1.0 / rs

    x = np.float32(np.pi / CUTOFF) * r
    s1 = jnp.sin(x)
    c1 = jnp.cos(x)

    c3 = (4.0 * c1 * c1 - 3.0) * c1
    fc = jnp.where(r > RMIN, 0.5 - 0.5 * (2.0 * c3 * c3 - 1.0), 1.0)

    pref = _RBF_PREF * inv * fc
    feats = [(pref * s1).reshape(1, FB)]
    s, c = s1, c1
    for _ in range(NUM_RADIAL - 1):
        s, c = s * c1 + c * s1, c * c1 - s * s1
        feats.append((pref * s).reshape(1, FB))

    fcrik = 0.5 + 0.5 * c1
    ux = (rx / rs).astype(jnp.bfloat16).astype(jnp.float32)
    uy = (ry / rs).astype(jnp.bfloat16).astype(jnp.float32)
    uz = (rz / rs).astype(jnp.bfloat16).astype(jnp.float32)
    cos3 = (
        ux[:, None, :] * ux[None, :, :]
        + uy[:, None, :] * uy[None, :, :]
        + uz[:, None, :] * uz[None, :, :]
    )
    il = jax.lax.broadcasted_iota(jnp.int32, (K, K, 1), 0)
    ik = jax.lax.broadcasted_iota(jnp.int32, (K, K, 1), 1)
    cos3 = jnp.where(il == ik, 0.0, cos3)

    p = jnp.exp(np.float32(-ETA) * cos3 * (cos3 + 2.0)) * fcrik[:, None, :]
    u = jnp.exp(_U_COEF * cos3)
    for m in range(NUM_3BODY):
        feats.append(
            (np.float32(_MU_SCALE[m]) * jnp.sum(p, axis=0)).reshape(1, FB))
        if m < NUM_3BODY - 1:
            p = p * u

    feats.append(jnp.ones((2, FB), dtype=jnp.float32))
    dmat = jnp.concatenate(feats, axis=0)
    pre = jnp.dot(w1t_ref[...], dmat.astype(jnp.bfloat16),
                  preferred_element_type=jnp.float32)
    h = jnp.tanh(pre)
    e = jnp.dot(w2t_ref[...], h.astype(jnp.bfloat16),
                preferred_element_type=jnp.float32) + b2_ref[0, 0]
    out_ref[...] = jnp.transpose(e.reshape(K, BN))


def kernel(elems, descriptors, beta, energy, rij, unique_i, unique_j,
           tag_i, tag_j, W1, b1, W2, b2):
    b1hi = b1.astype(jnp.bfloat16)
    b1lo = (b1 - b1hi.astype(jnp.float32)).astype(jnp.bfloat16)
    w1c = jnp.concatenate([W1.T.astype(jnp.bfloat16),
                           b1hi[:, None], b1lo[:, None]], axis=1)
    out = pl.pallas_call(
        _pair_nn_block,
        grid=(GRID,),
        in_specs=[
            pl.BlockSpec((3, K, BN), lambda i: (0, 0, i)),
            pl.BlockSpec((HIDDEN, N_DESC + 2), lambda i: (0, 0)),
            pl.BlockSpec((1, HIDDEN), lambda i: (0, 0)),
            pl.BlockSpec((1, 1), lambda i: (0, 0)),
        ],
        out_specs=pl.BlockSpec((BN, K), lambda i: (i, 0)),
        out_shape=jax.ShapeDtypeStruct((N, K), jnp.float32),
    )(rij.reshape(N, K, 3).transpose(2, 1, 0), w1c,
      W2.T.astype(jnp.bfloat16), b2.reshape(1, 1))
    return out.reshape(E, 1)
